# Initial kernel scaffold; baseline (speedup 1.0000x reference)
import jax
import jax.numpy as jnp
from jax.experimental import pallas as pl


def kernel(X_in):
    raise NotImplementedError("write your pallas kernel here")


# speedup vs baseline: 13.7133x; 13.7133x over previous
import functools

import jax
import jax.numpy as jnp
from jax import lax
from jax.experimental import pallas as pl
from jax.experimental.pallas import tpu as pltpu
from jax.experimental.pallas import tpu_sc as plsc


_NUM_CORES = 2
_NUM_SUBCORES = 16
_LANES = 16


def _make_scan_kernel(B, L, C):
    NW = _NUM_CORES * _NUM_SUBCORES
    CW = (B * C) // NW
    G = CW // _LANES
    LC = 512
    NT = L // LC
    WPB = C // CW

    mesh = plsc.VectorSubcoreMesh(
        core_axis_name="c", subcore_axis_name="s",
        num_cores=_NUM_CORES, num_subcores=_NUM_SUBCORES)

    @functools.partial(
        pl.kernel,
        out_type=jax.ShapeDtypeStruct((B, L, C), jnp.float32),
        mesh=mesh,
        compiler_params=pltpu.CompilerParams(use_tc_tiling_on_sc=False),
        scratch_types=[
            pltpu.VMEM((2, LC, CW), jnp.float32),
            pltpu.SemaphoreType.DMA,
            pltpu.SemaphoreType.DMA,
            pltpu.SemaphoreType.DMA,
            pltpu.SemaphoreType.DMA,
        ],
    )
    def scan_kernel(x_hbm, y_hbm, buf, in_sem0, in_sem1, out_sem0, out_sem1):
        cid = lax.axis_index("c")
        sid = lax.axis_index("s")
        wid = sid * _NUM_CORES + cid
        b = wid // WPB
        c0 = (wid % WPB) * CW
        in_sems = (in_sem0, in_sem1)
        out_sems = (out_sem0, out_sem1)

        def mk_in(t, slot):
            return pltpu.make_async_copy(
                x_hbm.at[b, pl.ds(t * LC, LC), pl.ds(c0, CW)],
                buf.at[slot],
                in_sems[slot],
            )

        def mk_out(t, slot):
            return pltpu.make_async_copy(
                buf.at[slot],
                y_hbm.at[b, pl.ds(t * LC, LC), pl.ds(c0, CW)],
                out_sems[slot],
            )

        def compute(slot, carries):
            def body(l, carries):
                new = []
                for g in range(G):
                    v = buf[slot, l, pl.ds(g * 16, 16)]
                    acc = carries[g] + v
                    buf[slot, l, pl.ds(g * 16, 16)] = acc
                    new.append(acc)
                return tuple(new)

            return lax.fori_loop(0, LC, body, carries)

        carries = tuple(jnp.zeros((16,), jnp.float32) for _ in range(G))
        in_copies = [None, None]
        out_copies = [None, None]
        in_copies[0] = mk_in(0, 0)
        in_copies[0].start()
        for t in range(NT):
            slot = t % 2
            nxt = (t + 1) % 2
            if t + 1 < NT:
                if out_copies[nxt] is not None:
                    out_copies[nxt].wait()
                in_copies[nxt] = mk_in(t + 1, nxt)
                in_copies[nxt].start()
            in_copies[slot].wait()
            carries = compute(slot, carries)
            out_copies[slot] = mk_out(t, slot)
            out_copies[slot].start()
        out_copies[(NT - 2) % 2].wait()
        out_copies[(NT - 1) % 2].wait()

    return scan_kernel


@jax.jit
def kernel(X_in):
    B, L, D, N = X_in.shape
    x = X_in.reshape(B, L, D * N)
    y = _make_scan_kernel(B, L, D * N)(x)
    return y.reshape(B, L, D, N)


# speedup vs baseline: 18.9335x; 1.3807x over previous
import functools

import jax
import jax.numpy as jnp
from jax import lax
from jax.experimental import pallas as pl
from jax.experimental.pallas import tpu as pltpu
from jax.experimental.pallas import tpu_sc as plsc

_NUM_CORES = 2
_NUM_SUBCORES = 16
_LANES = 16


def _make_scan_kernel(B, L, C):
    NW = _NUM_CORES * _NUM_SUBCORES
    CW = 128
    NWU = (B * C) // CW
    G = CW // _LANES
    LC = 256
    NT = L // LC
    WPB = C // CW

    mesh = plsc.VectorSubcoreMesh(
        core_axis_name="c", subcore_axis_name="s",
        num_cores=_NUM_CORES, num_subcores=_NUM_SUBCORES)

    @functools.partial(
        pl.kernel,
        out_type=jax.ShapeDtypeStruct((B, L, C), jnp.float32),
        mesh=mesh,
        scratch_types=[
            pltpu.VMEM((2, LC, CW), jnp.float32),
            pltpu.SemaphoreType.DMA,
            pltpu.SemaphoreType.DMA,
            pltpu.SemaphoreType.DMA,
            pltpu.SemaphoreType.DMA,
        ],
    )
    def scan_kernel(x_hbm, y_hbm, buf, in_sem0, in_sem1, out_sem0, out_sem1):
        cid = lax.axis_index("c")
        sid = lax.axis_index("s")
        wid = sid * _NUM_CORES + cid
        b = wid // WPB
        c0 = (wid % WPB) * CW
        in_sems = (in_sem0, in_sem1)
        out_sems = (out_sem0, out_sem1)

        def mk_in(t, slot):
            return pltpu.make_async_copy(
                x_hbm.at[b, pl.ds(t * LC, LC), pl.ds(c0, CW)],
                buf.at[slot],
                in_sems[slot],
            )

        def mk_out(t, slot):
            return pltpu.make_async_copy(
                buf.at[slot],
                y_hbm.at[b, pl.ds(t * LC, LC), pl.ds(c0, CW)],
                out_sems[slot],
            )

        def compute(slot, carries):
            def body(l, carries):
                new = []
                for g in range(G):
                    v = buf[slot, l, pl.ds(g * 16, 16)]
                    acc = carries[g] + v
                    buf[slot, l, pl.ds(g * 16, 16)] = acc
                    new.append(acc)
                return tuple(new)

            return lax.fori_loop(0, LC, body, carries)

        @pl.when(wid < NWU)
        def _():
            carries = tuple(jnp.zeros((16,), jnp.float32) for _ in range(G))
            in_copies = [None, None]
            out_copies = [None, None]
            in_copies[0] = mk_in(0, 0)
            in_copies[0].start()
            for t in range(NT):
                slot = t % 2
                nxt = (t + 1) % 2
                if t + 1 < NT:
                    if out_copies[nxt] is not None:
                        out_copies[nxt].wait()
                    in_copies[nxt] = mk_in(t + 1, nxt)
                    in_copies[nxt].start()
                in_copies[slot].wait()
                carries = compute(slot, carries)
                out_copies[slot] = mk_out(t, slot)
                out_copies[slot].start()
            out_copies[(NT - 2) % 2].wait()
            out_copies[(NT - 1) % 2].wait()

    return scan_kernel


@jax.jit
def kernel(X_in):
    B, L, D, N = X_in.shape
    x = X_in.reshape(B, L, D * N)
    y = _make_scan_kernel(B, L, D * N)(x)
    return y.reshape(B, L, D, N)


# speedup vs baseline: 18.9566x; 1.0012x over previous
import functools

import jax
import jax.numpy as jnp
from jax import lax
from jax.experimental import pallas as pl
from jax.experimental.pallas import tpu as pltpu
from jax.experimental.pallas import tpu_sc as plsc

_NUM_CORES = 2
_NUM_SUBCORES = 16
_LANES = 16


def _make_scan_kernel(B, L, C):
    NW = _NUM_CORES * _NUM_SUBCORES
    CW = 128
    NWU = (B * C) // CW
    G = CW // _LANES
    LC = 256
    NT = L // LC
    WPB = C // CW

    mesh = plsc.VectorSubcoreMesh(
        core_axis_name="c", subcore_axis_name="s",
        num_cores=_NUM_CORES, num_subcores=_NUM_SUBCORES)

    @functools.partial(
        pl.kernel,
        out_type=jax.ShapeDtypeStruct((B, L, WPB, CW), jnp.float32),
        mesh=mesh,
        scratch_types=[
            pltpu.VMEM((2, LC, 1, CW), jnp.float32),
            pltpu.SemaphoreType.DMA,
            pltpu.SemaphoreType.DMA,
            pltpu.SemaphoreType.DMA,
            pltpu.SemaphoreType.DMA,
        ],
    )
    def scan_kernel(x_hbm, y_hbm, buf, in_sem0, in_sem1, out_sem0, out_sem1):
        cid = lax.axis_index("c")
        sid = lax.axis_index("s")
        wid = sid * _NUM_CORES + cid
        b = wid // WPB
        slab = wid % WPB
        in_sems = (in_sem0, in_sem1)
        out_sems = (out_sem0, out_sem1)

        def mk_in(t, slot):
            return pltpu.make_async_copy(
                x_hbm.at[b, pl.ds(t * LC, LC), pl.ds(slab, 1)],
                buf.at[slot],
                in_sems[slot],
            )

        def mk_out(t, slot):
            return pltpu.make_async_copy(
                buf.at[slot],
                y_hbm.at[b, pl.ds(t * LC, LC), pl.ds(slab, 1)],
                out_sems[slot],
            )

        def compute(slot, carries):
            def body(l, carries):
                new = []
                for g in range(G):
                    v = buf[slot, l, 0, pl.ds(g * 16, 16)]
                    acc = carries[g] + v
                    buf[slot, l, 0, pl.ds(g * 16, 16)] = acc
                    new.append(acc)
                return tuple(new)

            return lax.fori_loop(0, LC, body, carries)

        @pl.when(wid < NWU)
        def _():
            carries = tuple(jnp.zeros((16,), jnp.float32) for _ in range(G))
            in_copies = [None, None]
            out_copies = [None, None]
            in_copies[0] = mk_in(0, 0)
            in_copies[0].start()
            for t in range(NT):
                slot = t % 2
                nxt = (t + 1) % 2
                if t + 1 < NT:
                    if out_copies[nxt] is not None:
                        out_copies[nxt].wait()
                    in_copies[nxt] = mk_in(t + 1, nxt)
                    in_copies[nxt].start()
                in_copies[slot].wait()
                carries = compute(slot, carries)
                out_copies[slot] = mk_out(t, slot)
                out_copies[slot].start()
            out_copies[(NT - 2) % 2].wait()
            out_copies[(NT - 1) % 2].wait()

    return scan_kernel


@jax.jit
def kernel(X_in):
    B, L, D, N = X_in.shape
    x = X_in.reshape(B, L, (D * N) // 128, 128)
    y = _make_scan_kernel(B, L, D * N)(x)
    return y.reshape(B, L, D, N)


# speedup vs baseline: 35.5897x; 1.8774x over previous
import functools

import jax
import jax.numpy as jnp
from jax import lax
from jax.experimental import pallas as pl
from jax.experimental.pallas import tpu as pltpu
from jax.experimental.pallas import tpu_sc as plsc

_NUM_CORES = 2
_NUM_SUBCORES = 16
_LANES = 16


def _make_scan_kernel(R, L):
    NW = _NUM_CORES * _NUM_SUBCORES
    RW = R // NW
    RG = _LANES
    NG = RW // RG
    CC = 2048
    NC = L // CC
    NT = NG * NC

    mesh = plsc.VectorSubcoreMesh(
        core_axis_name="c", subcore_axis_name="s",
        num_cores=_NUM_CORES, num_subcores=_NUM_SUBCORES)

    @functools.partial(
        pl.kernel,
        out_type=jax.ShapeDtypeStruct((R, L), jnp.float32),
        mesh=mesh,
        compiler_params=pltpu.CompilerParams(needs_layout_passes=False),
        scratch_types=[
            pltpu.VMEM((2, RG, CC), jnp.float32),
            pltpu.SemaphoreType.DMA,
            pltpu.SemaphoreType.DMA,
            pltpu.SemaphoreType.DMA,
            pltpu.SemaphoreType.DMA,
        ],
    )
    def scan_kernel(x_hbm, y_hbm, buf, in_sem0, in_sem1, out_sem0, out_sem1):
        cid = lax.axis_index("c")
        sid = lax.axis_index("s")
        wid = sid * _NUM_CORES + cid
        r0 = wid * RW
        in_sems = (in_sem0, in_sem1)
        out_sems = (out_sem0, out_sem1)
        last_lane = jnp.full((_LANES,), _LANES - 1, jnp.int32)

        def mk_in(t, slot):
            g, c = t // NC, t % NC
            return pltpu.make_async_copy(
                x_hbm.at[pl.ds(r0 + g * RG, RG), pl.ds(c * CC, CC)],
                buf.at[slot],
                in_sems[slot],
            )

        def mk_out(t, slot):
            g, c = t // NC, t % NC
            return pltpu.make_async_copy(
                buf.at[slot],
                y_hbm.at[pl.ds(r0 + g * RG, RG), pl.ds(c * CC, CC)],
                out_sems[slot],
            )

        def compute(slot, accs):
            def body(k, accs):
                col = k * _LANES
                new = []
                for r in range(RG):
                    v = buf[slot, r, pl.ds(col, _LANES)]
                    s = plsc.cumsum(v)
                    tot = lax.gather(
                        s, last_lane[:, None],
                        lax.GatherDimensionNumbers(
                            offset_dims=(), collapsed_slice_dims=(0,),
                            start_index_map=(0,)),
                        (1,), mode=lax.GatherScatterMode.PROMISE_IN_BOUNDS)
                    buf[slot, r, pl.ds(col, _LANES)] = s + accs[r]
                    new.append(accs[r] + tot)
                return tuple(new)

            return lax.fori_loop(0, CC // _LANES, body, accs)

        in_copies = [None, None]
        out_copies = [None, None]
        in_copies[0] = mk_in(0, 0)
        in_copies[0].start()
        accs = tuple(jnp.zeros((_LANES,), jnp.float32) for _ in range(RG))
        for t in range(NT):
            slot = t % 2
            nxt = (t + 1) % 2
            if t + 1 < NT:
                if out_copies[nxt] is not None:
                    out_copies[nxt].wait()
                in_copies[nxt] = mk_in(t + 1, nxt)
                in_copies[nxt].start()
            in_copies[slot].wait()
            if t % NC == 0:
                accs = tuple(
                    jnp.zeros((_LANES,), jnp.float32) for _ in range(RG))
            accs = compute(slot, accs)
            out_copies[slot] = mk_out(t, slot)
            out_copies[slot].start()
        out_copies[(NT - 2) % 2].wait()
        out_copies[(NT - 1) % 2].wait()

    return scan_kernel


@jax.jit
def kernel(X_in):
    B, L, D, N = X_in.shape
    xt = jnp.transpose(X_in, (0, 2, 3, 1)).reshape(B * D * N, L)
    y = _make_scan_kernel(B * D * N, L)(xt)
    return jnp.transpose(y.reshape(B, D, N, L), (0, 3, 1, 2))


# speedup vs baseline: 36.9425x; 1.0380x over previous
import functools

import jax
import jax.numpy as jnp
from jax import lax
from jax.experimental import pallas as pl
from jax.experimental.pallas import tpu as pltpu
from jax.experimental.pallas import tpu_sc as plsc

_NUM_CORES = 2
_NUM_SUBCORES = 16
_LANES = 16


def _make_scan_kernel(R, L):
    NW = _NUM_CORES * _NUM_SUBCORES
    RW = R // NW
    RG = _LANES
    NG = RW // RG
    CC = 2048
    NC = L // CC
    NT = NG * NC

    mesh = plsc.VectorSubcoreMesh(
        core_axis_name="c", subcore_axis_name="s",
        num_cores=_NUM_CORES, num_subcores=_NUM_SUBCORES)

    @functools.partial(
        pl.kernel,
        out_type=jax.ShapeDtypeStruct((R, L), jnp.float32),
        mesh=mesh,
        compiler_params=pltpu.CompilerParams(needs_layout_passes=False),
        scratch_types=[
            pltpu.VMEM((3, RG, CC), jnp.float32),
            pltpu.SemaphoreType.DMA,
            pltpu.SemaphoreType.DMA,
            pltpu.SemaphoreType.DMA,
            pltpu.SemaphoreType.DMA,
            pltpu.SemaphoreType.DMA,
            pltpu.SemaphoreType.DMA,
        ],
    )
    def scan_kernel(x_hbm, y_hbm, buf, in_sem0, in_sem1, in_sem2,
                    out_sem0, out_sem1, out_sem2):
        cid = lax.axis_index("c")
        sid = lax.axis_index("s")
        wid = sid * _NUM_CORES + cid
        r0 = wid * RW
        in_sems = (in_sem0, in_sem1, in_sem2)
        out_sems = (out_sem0, out_sem1, out_sem2)
        last_lane = jnp.full((_LANES,), _LANES - 1, jnp.int32)

        def mk_in(t, slot):
            g, c = t // NC, t % NC
            return pltpu.make_async_copy(
                x_hbm.at[pl.ds(r0 + g * RG, RG), pl.ds(c * CC, CC)],
                buf.at[slot],
                in_sems[slot],
            )

        def mk_out(t, slot):
            g, c = t // NC, t % NC
            return pltpu.make_async_copy(
                buf.at[slot],
                y_hbm.at[pl.ds(r0 + g * RG, RG), pl.ds(c * CC, CC)],
                out_sems[slot],
            )

        def compute(slot, accs):
            def body(k, accs):
                col = k * _LANES
                new = []
                for r in range(RG):
                    v = buf[slot, r, pl.ds(col, _LANES)]
                    s = plsc.cumsum(v)
                    tot = lax.gather(
                        s, last_lane[:, None],
                        lax.GatherDimensionNumbers(
                            offset_dims=(), collapsed_slice_dims=(0,),
                            start_index_map=(0,)),
                        (1,), mode=lax.GatherScatterMode.PROMISE_IN_BOUNDS)
                    buf[slot, r, pl.ds(col, _LANES)] = s + accs[r]
                    new.append(accs[r] + tot)
                return tuple(new)

            return lax.fori_loop(0, CC // _LANES, body, accs)

        NB = 3
        in_copies = [None] * NB
        out_copies = [None] * NB
        for p in range(min(2, NT)):
            in_copies[p] = mk_in(p, p)
            in_copies[p].start()
        accs = tuple(jnp.zeros((_LANES,), jnp.float32) for _ in range(RG))
        for t in range(NT):
            slot = t % NB
            if t + 2 < NT:
                nxt = (t + 2) % NB
                if out_copies[nxt] is not None:
                    out_copies[nxt].wait()
                in_copies[nxt] = mk_in(t + 2, nxt)
                in_copies[nxt].start()
            in_copies[slot].wait()
            if t % NC == 0:
                accs = tuple(
                    jnp.zeros((_LANES,), jnp.float32) for _ in range(RG))
            accs = compute(slot, accs)
            out_copies[slot] = mk_out(t, slot)
            out_copies[slot].start()
        for p in range(min(NB, NT)):
            out_copies[(NT - 1 - p) % NB].wait()

    return scan_kernel


@jax.jit
def kernel(X_in):
    B, L, D, N = X_in.shape
    xt = jnp.transpose(X_in, (0, 2, 3, 1)).reshape(B * D * N, L)
    y = _make_scan_kernel(B * D * N, L)(xt)
    return jnp.transpose(y.reshape(B, D, N, L), (0, 3, 1, 2))


# speedup vs baseline: 37.6097x; 1.0181x over previous
import functools

import jax
import jax.numpy as jnp
from jax import lax
from jax.experimental import pallas as pl
from jax.experimental.pallas import tpu as pltpu
from jax.experimental.pallas import tpu_sc as plsc

_NUM_CORES = 2
_NUM_SUBCORES = 16
_LANES = 16


def _make_scan_kernel(R, L):
    NW = _NUM_CORES * _NUM_SUBCORES
    RW = R // NW
    RG = _LANES
    NG = RW // RG
    CC = 2048
    NC = L // CC
    NT = NG * NC

    mesh = plsc.VectorSubcoreMesh(
        core_axis_name="c", subcore_axis_name="s",
        num_cores=_NUM_CORES, num_subcores=_NUM_SUBCORES)

    @functools.partial(
        pl.kernel,
        out_type=jax.ShapeDtypeStruct((R, L), jnp.float32),
        mesh=mesh,
        compiler_params=pltpu.CompilerParams(needs_layout_passes=False),
        scratch_types=[
            pltpu.VMEM((3, RG, CC), jnp.float32),
            pltpu.SemaphoreType.DMA,
            pltpu.SemaphoreType.DMA,
            pltpu.SemaphoreType.DMA,
            pltpu.SemaphoreType.DMA,
            pltpu.SemaphoreType.DMA,
            pltpu.SemaphoreType.DMA,
        ],
    )
    def scan_kernel(x_hbm, y_hbm, buf, in_sem0, in_sem1, in_sem2,
                    out_sem0, out_sem1, out_sem2):
        cid = lax.axis_index("c")
        sid = lax.axis_index("s")
        wid = sid * _NUM_CORES + cid
        r0 = wid * RW
        in_sems = (in_sem0, in_sem1, in_sem2)
        out_sems = (out_sem0, out_sem1, out_sem2)
        last_lane = jnp.full((_LANES,), _LANES - 1, jnp.int32)

        def mk_in(t, slot):
            g, c = t // NC, t % NC
            return pltpu.make_async_copy(
                x_hbm.at[pl.ds(r0 + g * RG, RG), pl.ds(c * CC, CC)],
                buf.at[slot],
                in_sems[slot],
            )

        def mk_out(t, slot):
            g, c = t // NC, t % NC
            return pltpu.make_async_copy(
                buf.at[slot],
                y_hbm.at[pl.ds(r0 + g * RG, RG), pl.ds(c * CC, CC)],
                out_sems[slot],
            )

        def compute(slot, accs):
            def body(k, accs):
                col = k * _LANES
                new = []
                for r in range(RG):
                    v = buf[slot, r, pl.ds(col, _LANES)]
                    s = plsc.cumsum(v)
                    out = s + accs[r]
                    buf[slot, r, pl.ds(col, _LANES)] = out
                    new.append(jnp.full((_LANES,), out[_LANES - 1],
                                        jnp.float32))
                return tuple(new)

            return lax.fori_loop(0, CC // _LANES, body, accs)

        NB = 3
        in_copies = [None] * NB
        out_copies = [None] * NB
        for p in range(min(2, NT)):
            in_copies[p] = mk_in(p, p)
            in_copies[p].start()
        accs = tuple(jnp.zeros((_LANES,), jnp.float32) for _ in range(RG))
        for t in range(NT):
            slot = t % NB
            if t + 2 < NT:
                nxt = (t + 2) % NB
                if out_copies[nxt] is not None:
                    out_copies[nxt].wait()
                in_copies[nxt] = mk_in(t + 2, nxt)
                in_copies[nxt].start()
            in_copies[slot].wait()
            if t % NC == 0:
                accs = tuple(
                    jnp.zeros((_LANES,), jnp.float32) for _ in range(RG))
            accs = compute(slot, accs)
            out_copies[slot] = mk_out(t, slot)
            out_copies[slot].start()
        for p in range(min(NB, NT)):
            out_copies[(NT - 1 - p) % NB].wait()

    return scan_kernel


@jax.jit
def kernel(X_in):
    B, L, D, N = X_in.shape
    xt = jnp.transpose(X_in, (0, 2, 3, 1)).reshape(B * D * N, L)
    y = _make_scan_kernel(B * D * N, L)(xt)
    return jnp.transpose(y.reshape(B, D, N, L), (0, 3, 1, 2))
